# Initial kernel scaffold; baseline (speedup 1.0000x reference)
import jax
import jax.numpy as jnp
from jax.experimental import pallas as pl


def kernel(x, edge_index, conv_Wr, conv_Wi, out_W, out_b):
    raise NotImplementedError("write your pallas kernel here")


# speedup vs baseline: 4.8961x; 4.8961x over previous
import functools

import jax
import jax.numpy as jnp
from jax import lax
from jax.experimental import pallas as pl
from jax.experimental.pallas import tpu as pltpu
from jax.experimental.pallas import tpu_sc as plsc

N = 10000
NP = 10240
E = 320000
D = 128
NC = 2
NS = 16
EPT = E // NS
K = 80
NCHUNK = EPT // K
RPT = NP // NS
ZROWS = 128
NZ = RPT // ZROWS


def _sc_agg_body(t_hbm, src_hbm, dst_hbm, z_hbm,
                 srcv, dstv, rows, zbuf, acc, sem):
    c = lax.axis_index("c")
    s = lax.axis_index("s")

    def _zero_row(r, _):
        for u in range(D // 16):
            zbuf[r, pl.ds(u * 16, 16)] = jnp.zeros((16,), jnp.float32)
        return 0
    lax.fori_loop(0, ZROWS, _zero_row, 0)
    for q in range(NZ):
        pltpu.sync_copy(zbuf, acc.at[pl.ds(s * RPT + q * ZROWS, ZROWS)])
    plsc.subcore_barrier()

    ebase = c * E + s * EPT
    dbase = s * EPT

    def _chunk(j, _):
        pltpu.sync_copy(src_hbm.at[pl.ds(ebase + j * K, K)], srcv)
        pltpu.sync_copy(dst_hbm.at[pl.ds(dbase + j * K, K)], dstv)
        pltpu.async_copy(t_hbm.at[srcv], rows, sem).wait()
        pltpu.sync_copy(rows, acc.at[dstv], add=True)
        return 0
    lax.fori_loop(0, NCHUNK, _chunk, 0)
    plsc.subcore_barrier()

    pltpu.sync_copy(acc.at[pl.ds(s * RPT, RPT)],
                    z_hbm.at[pl.ds(c * NP + s * RPT, RPT)])


@functools.cache
def _get_sc_agg():
    return pl.kernel(
        _sc_agg_body,
        out_type=jax.ShapeDtypeStruct((2 * NP, D), jnp.float32),
        mesh=plsc.VectorSubcoreMesh(core_axis_name="c", subcore_axis_name="s",
                                    num_cores=NC, num_subcores=NS),
        scratch_types=[
            pltpu.VMEM((K,), jnp.int32),
            pltpu.VMEM((K,), jnp.int32),
            pltpu.VMEM((K, D), jnp.float32),
            pltpu.VMEM((ZROWS, D), jnp.float32),
            pltpu.VMEM_SHARED((NP, D), jnp.float32),
            pltpu.SemaphoreType.DMA,
        ],
    )


def _sc_agg(t, src_flat, dst):
    return _get_sc_agg()(t, src_flat, dst)


_B = 1000
_GRID = N // _B


def _scale_body(zdeg_ref, dinv_ref, dinv2_ref):
    deg = jnp.maximum(zdeg_ref[...], 1.0)
    dinv2_ref[...] = 1.0 / deg
    dinv_ref[...] = lax.rsqrt(deg)


def _tc_scale(zdeg):
    return pl.pallas_call(
        _scale_body,
        grid=(_GRID,),
        in_specs=[pl.BlockSpec((_B, D), lambda i: (i, 0))],
        out_specs=[pl.BlockSpec((_B, D), lambda i: (i, 0)),
                   pl.BlockSpec((_B, D), lambda i: (i, 0))],
        out_shape=[jax.ShapeDtypeStruct((N, D), jnp.float32),
                   jax.ShapeDtypeStruct((N, D), jnp.float32)],
    )(zdeg)


def _layer0_body(x_ref, wr_ref, wi_ref, dinv_ref, gr_ref, gi_ref):
    x = x_ref[...]
    d = dinv_ref[...]
    gr_ref[...] = d * jnp.dot(x, wr_ref[...], preferred_element_type=jnp.float32)
    gi_ref[...] = d * jnp.dot(x, wi_ref[...], preferred_element_type=jnp.float32)


def _tc_layer0(x, wr, wi, dinv_b):
    return pl.pallas_call(
        _layer0_body,
        grid=(_GRID,),
        in_specs=[pl.BlockSpec((_B, D), lambda i: (i, 0)),
                  pl.BlockSpec((D, D), lambda i: (0, 0)),
                  pl.BlockSpec((D, D), lambda i: (0, 0)),
                  pl.BlockSpec((_B, D), lambda i: (i, 0))],
        out_specs=[pl.BlockSpec((_B, D), lambda i: (i, 0)),
                   pl.BlockSpec((_B, D), lambda i: (i, 0))],
        out_shape=[jax.ShapeDtypeStruct((N, D), jnp.float32),
                   jax.ShapeDtypeStruct((N, D), jnp.float32)],
    )(x, wr, wi, dinv_b)


def _layer_body(zr_ref, zi_ref, wr_ref, wi_ref, d2_ref, gr_ref, gi_ref):
    wr = wr_ref[...]
    wi = wi_ref[...]
    wra = 0.5 * (wr - wr.T)
    wia = 0.5 * (wi + wi.T)
    zr = zr_ref[...]
    zi = zi_ref[...]
    d2 = d2_ref[...]
    gr_ref[...] = d2 * (jnp.dot(zr, wra, preferred_element_type=jnp.float32)
                        - jnp.dot(zi, wia, preferred_element_type=jnp.float32))
    gi_ref[...] = d2 * (jnp.dot(zr, wia, preferred_element_type=jnp.float32)
                        + jnp.dot(zi, wra, preferred_element_type=jnp.float32))


def _tc_layer2(zr, zi, wr, wi, dinv2_b):
    return pl.pallas_call(
        _layer_body,
        grid=(_GRID,),
        in_specs=[pl.BlockSpec((_B, D), lambda i: (i, 0)),
                  pl.BlockSpec((_B, D), lambda i: (i, 0)),
                  pl.BlockSpec((D, D), lambda i: (0, 0)),
                  pl.BlockSpec((D, D), lambda i: (0, 0)),
                  pl.BlockSpec((_B, D), lambda i: (i, 0))],
        out_specs=[pl.BlockSpec((_B, D), lambda i: (i, 0)),
                   pl.BlockSpec((_B, D), lambda i: (i, 0))],
        out_shape=[jax.ShapeDtypeStruct((N, D), jnp.float32),
                   jax.ShapeDtypeStruct((N, D), jnp.float32)],
    )(zr, zi, wr, wi, dinv2_b)


def _final_body(zr_ref, dinv_ref, w_ref, b_ref, out_ref):
    xr = dinv_ref[...] * zr_ref[...]
    out_ref[...] = (jnp.dot(xr, w_ref[...], preferred_element_type=jnp.float32)
                    + b_ref[...])


def _tc_final(z, dinv_b, out_w, out_b2d):
    return pl.pallas_call(
        _final_body,
        grid=(_GRID,),
        in_specs=[pl.BlockSpec((_B, D), lambda i: (i, 0)),
                  pl.BlockSpec((_B, D), lambda i: (i, 0)),
                  pl.BlockSpec((D, D), lambda i: (0, 0)),
                  pl.BlockSpec((1, D), lambda i: (0, 0))],
        out_specs=pl.BlockSpec((_B, D), lambda i: (i, 0)),
        out_shape=jax.ShapeDtypeStruct((N, D), jnp.float32),
    )(z, dinv_b, out_w, out_b2d)


def kernel(x, edge_index, conv_Wr, conv_Wi, out_W, out_b):
    src = edge_index[0]
    dst = edge_index[1]
    src_flat = jnp.concatenate([src, src + NP])
    pad = jnp.zeros((NP - N, D), jnp.float32)
    ones_t = jnp.ones((NP + N, D), jnp.float32)

    zdeg = _sc_agg(ones_t, src_flat, dst)
    dinv_b, dinv2_b = _tc_scale(zdeg[:N])

    gr, gi = _tc_layer0(x, conv_Wr[0], conv_Wi[0], dinv_b)
    nlayers = conv_Wr.shape[0]
    for i in range(1, nlayers):
        t = jnp.concatenate([gr, pad, gi], axis=0)
        z = _sc_agg(t, src_flat, dst)
        gr, gi = _tc_layer2(z[:N], z[NP:NP + N], conv_Wr[i], conv_Wi[i], dinv2_b)

    t = jnp.concatenate([gr, pad, gi], axis=0)
    z = _sc_agg(t, src_flat, dst)
    return _tc_final(z[:N], dinv_b, out_W, out_b.reshape(1, D))


# speedup vs baseline: 12.2151x; 2.4949x over previous
import functools

import jax
import jax.numpy as jnp
from jax import lax
from jax.experimental import pallas as pl
from jax.experimental.pallas import tpu as pltpu
from jax.experimental.pallas import tpu_sc as plsc

N = 10000
NP = 10240
E = 320000
D = 128
NC = 2
NS = 16
EPT = E // NS
K = 80
NCHUNK = EPT // K
RPT = NP // NS
ZROWS = 32
NZ = RPT // ZROWS


def _sc_agg_body(t_hbm, src_hbm, dst_hbm, z_hbm,
                 srcv, dstk0, dstk1, rows0, rows1, zbuf, acc,
                 sem0, sem1, semd0, semd1):
    c = lax.axis_index("c")
    s = lax.axis_index("s")

    def _zero_row(r, _):
        for u in range(D // 16):
            zbuf[r, pl.ds(u * 16, 16)] = jnp.zeros((16,), jnp.float32)
        return 0
    lax.fori_loop(0, ZROWS, _zero_row, 0)
    for q in range(NZ):
        pltpu.sync_copy(zbuf, acc.at[pl.ds(s * RPT + q * ZROWS, ZROWS)])

    pltpu.sync_copy(src_hbm.at[pl.ds(c * E + s * EPT, EPT)], srcv)
    plsc.subcore_barrier()

    dbase = s * EPT

    def _gather(j, rows, sem):
        pltpu.async_copy(t_hbm.at[srcv.at[pl.ds(j * K, K)]], rows, sem)

    def _wait_g(j, rows, sem):
        pltpu.make_async_copy(t_hbm.at[srcv.at[pl.ds(j * K, K)]], rows,
                              sem).wait()

    def _dload(j, dk, semd):
        pltpu.async_copy(dst_hbm.at[pl.ds(dbase + j * K, K)], dk, semd)

    def _wait_d(j, dk, semd):
        pltpu.make_async_copy(dst_hbm.at[pl.ds(dbase + j * K, K)], dk,
                              semd).wait()

    _dload(0, dstk0, semd0)
    _dload(1, dstk1, semd1)
    _gather(0, rows0, sem0)
    _gather(1, rows1, sem1)

    def _pair(jj, _):
        j = 2 * jj
        _wait_g(j, rows0, sem0)
        _wait_d(j, dstk0, semd0)
        pltpu.sync_copy(rows0, acc.at[dstk0], add=True)
        _dload(j + 2, dstk0, semd0)
        _gather(j + 2, rows0, sem0)
        _wait_g(j + 1, rows1, sem1)
        _wait_d(j + 1, dstk1, semd1)
        pltpu.sync_copy(rows1, acc.at[dstk1], add=True)
        _dload(j + 3, dstk1, semd1)
        _gather(j + 3, rows1, sem1)
        return 0
    lax.fori_loop(0, NCHUNK // 2 - 1, _pair, 0)

    _wait_g(NCHUNK - 2, rows0, sem0)
    _wait_d(NCHUNK - 2, dstk0, semd0)
    pltpu.sync_copy(rows0, acc.at[dstk0], add=True)
    _wait_g(NCHUNK - 1, rows1, sem1)
    _wait_d(NCHUNK - 1, dstk1, semd1)
    pltpu.sync_copy(rows1, acc.at[dstk1], add=True)

    plsc.subcore_barrier()
    pltpu.sync_copy(acc.at[pl.ds(s * RPT, RPT)],
                    z_hbm.at[pl.ds(c * NP + s * RPT, RPT)])


@functools.cache
def _get_sc_agg():
    return pl.kernel(
        _sc_agg_body,
        out_type=jax.ShapeDtypeStruct((2 * NP, D), jnp.float32),
        mesh=plsc.VectorSubcoreMesh(core_axis_name="c", subcore_axis_name="s",
                                    num_cores=NC, num_subcores=NS),
        scratch_types=[
            pltpu.VMEM((EPT,), jnp.int32),
            pltpu.VMEM((K,), jnp.int32),
            pltpu.VMEM((K,), jnp.int32),
            pltpu.VMEM((K, D), jnp.float32),
            pltpu.VMEM((K, D), jnp.float32),
            pltpu.VMEM((ZROWS, D), jnp.float32),
            pltpu.VMEM_SHARED((NP, D), jnp.float32),
            pltpu.SemaphoreType.DMA,
            pltpu.SemaphoreType.DMA,
            pltpu.SemaphoreType.DMA,
            pltpu.SemaphoreType.DMA,
        ],
    )


def _sc_agg(t, src3d, dst3d):
    return _get_sc_agg()(t, src3d, dst3d)


_B = 1000
_GRID = N // _B


def _scale_body(zdeg_ref, dinv_ref, dinv2_ref):
    deg = jnp.maximum(zdeg_ref[...], 1.0)
    dinv2_ref[...] = 1.0 / deg
    dinv_ref[...] = lax.rsqrt(deg)


def _tc_scale(zdeg):
    return pl.pallas_call(
        _scale_body,
        grid=(_GRID,),
        in_specs=[pl.BlockSpec((_B, D), lambda i: (i, 0))],
        out_specs=[pl.BlockSpec((_B, D), lambda i: (i, 0)),
                   pl.BlockSpec((_B, D), lambda i: (i, 0))],
        out_shape=[jax.ShapeDtypeStruct((N, D), jnp.float32),
                   jax.ShapeDtypeStruct((N, D), jnp.float32)],
    )(zdeg)


def _layer0_body(x_ref, wr_ref, wi_ref, dinv_ref, gr_ref, gi_ref):
    x = x_ref[...]
    d = dinv_ref[...]
    gr_ref[...] = d * jnp.dot(x, wr_ref[...], preferred_element_type=jnp.float32)
    gi_ref[...] = d * jnp.dot(x, wi_ref[...], preferred_element_type=jnp.float32)


def _tc_layer0(x, wr, wi, dinv_b):
    return pl.pallas_call(
        _layer0_body,
        grid=(_GRID,),
        in_specs=[pl.BlockSpec((_B, D), lambda i: (i, 0)),
                  pl.BlockSpec((D, D), lambda i: (0, 0)),
                  pl.BlockSpec((D, D), lambda i: (0, 0)),
                  pl.BlockSpec((_B, D), lambda i: (i, 0))],
        out_specs=[pl.BlockSpec((_B, D), lambda i: (i, 0)),
                   pl.BlockSpec((_B, D), lambda i: (i, 0))],
        out_shape=[jax.ShapeDtypeStruct((N, D), jnp.float32),
                   jax.ShapeDtypeStruct((N, D), jnp.float32)],
    )(x, wr, wi, dinv_b)


def _layer_body(zr_ref, zi_ref, wr_ref, wi_ref, d2_ref, gr_ref, gi_ref):
    wr = wr_ref[...]
    wi = wi_ref[...]
    wra = 0.5 * (wr - wr.T)
    wia = 0.5 * (wi + wi.T)
    zr = zr_ref[...]
    zi = zi_ref[...]
    d2 = d2_ref[...]
    gr_ref[...] = d2 * (jnp.dot(zr, wra, preferred_element_type=jnp.float32)
                        - jnp.dot(zi, wia, preferred_element_type=jnp.float32))
    gi_ref[...] = d2 * (jnp.dot(zr, wia, preferred_element_type=jnp.float32)
                        + jnp.dot(zi, wra, preferred_element_type=jnp.float32))


def _tc_layer2(zr, zi, wr, wi, dinv2_b):
    return pl.pallas_call(
        _layer_body,
        grid=(_GRID,),
        in_specs=[pl.BlockSpec((_B, D), lambda i: (i, 0)),
                  pl.BlockSpec((_B, D), lambda i: (i, 0)),
                  pl.BlockSpec((D, D), lambda i: (0, 0)),
                  pl.BlockSpec((D, D), lambda i: (0, 0)),
                  pl.BlockSpec((_B, D), lambda i: (i, 0))],
        out_specs=[pl.BlockSpec((_B, D), lambda i: (i, 0)),
                   pl.BlockSpec((_B, D), lambda i: (i, 0))],
        out_shape=[jax.ShapeDtypeStruct((N, D), jnp.float32),
                   jax.ShapeDtypeStruct((N, D), jnp.float32)],
    )(zr, zi, wr, wi, dinv2_b)


def _final_body(zr_ref, dinv_ref, w_ref, b_ref, out_ref):
    xr = dinv_ref[...] * zr_ref[...]
    out_ref[...] = (jnp.dot(xr, w_ref[...], preferred_element_type=jnp.float32)
                    + b_ref[...])


def _tc_final(z, dinv_b, out_w, out_b2d):
    return pl.pallas_call(
        _final_body,
        grid=(_GRID,),
        in_specs=[pl.BlockSpec((_B, D), lambda i: (i, 0)),
                  pl.BlockSpec((_B, D), lambda i: (i, 0)),
                  pl.BlockSpec((D, D), lambda i: (0, 0)),
                  pl.BlockSpec((1, D), lambda i: (0, 0))],
        out_specs=pl.BlockSpec((_B, D), lambda i: (i, 0)),
        out_shape=jax.ShapeDtypeStruct((N, D), jnp.float32),
    )(z, dinv_b, out_w, out_b2d)


def kernel(x, edge_index, conv_Wr, conv_Wi, out_W, out_b):
    src = edge_index[0]
    dst = edge_index[1]
    src3d = jnp.concatenate([src, src + NP])
    dst3d = dst
    pad = jnp.zeros((NP - N, D), jnp.float32)
    ones_t = jnp.ones((NP + N, D), jnp.float32)

    zdeg = _sc_agg(ones_t, src3d, dst3d)
    dinv_b, dinv2_b = _tc_scale(zdeg[:N])

    gr, gi = _tc_layer0(x, conv_Wr[0], conv_Wi[0], dinv_b)
    nlayers = conv_Wr.shape[0]
    for i in range(1, nlayers):
        t = jnp.concatenate([gr, pad, gi], axis=0)
        z = _sc_agg(t, src3d, dst3d)
        gr, gi = _tc_layer2(z[:N], z[NP:NP + N], conv_Wr[i], conv_Wi[i], dinv2_b)

    t = jnp.concatenate([gr, pad, gi], axis=0)
    z = _sc_agg(t, src3d, dst3d)
    return _tc_final(z[:N], dinv_b, out_W, out_b.reshape(1, D))


# speedup vs baseline: 13.9804x; 1.1445x over previous
import functools

import jax
import jax.numpy as jnp
from jax import lax
from jax.experimental import pallas as pl
from jax.experimental.pallas import tpu as pltpu
from jax.experimental.pallas import tpu_sc as plsc

N = 10000
NP = 10240
E = 320000
D = 128
NC = 2
NS = 16
EPT = E // NS
K = 80
NCHUNK = EPT // K
RPT = NP // NS
ZROWS = 32
NZ = RPT // ZROWS


def _make_agg_body(dw, ept, split_edges):
    nchunk = ept // K
    nz = RPT // ZROWS

    def body(t_hbm, src_hbm, dst_hbm, z_hbm,
             srcv, dstk0, dstk1, rows0, rows1, zbuf, acc,
             sem0, sem1, semd0, semd1):
        c = lax.axis_index("c")
        s = lax.axis_index("s")

        def _zero_row(r, _):
            for u in range(dw // 16):
                zbuf[r, pl.ds(u * 16, 16)] = jnp.zeros((16,), jnp.float32)
            return 0
        lax.fori_loop(0, ZROWS, _zero_row, 0)
        for q in range(nz):
            pltpu.sync_copy(zbuf, acc.at[pl.ds(s * RPT + q * ZROWS, ZROWS)])

        if split_edges:
            ebase = c * (E // 2) + s * ept
            dbase = ebase
        else:
            ebase = c * E + s * ept
            dbase = s * ept

        pltpu.sync_copy(src_hbm.at[pl.ds(ebase, ept)], srcv)
        plsc.subcore_barrier()

        def _gather(j, rows, sem):
            pltpu.async_copy(t_hbm.at[srcv.at[pl.ds(j * K, K)]], rows, sem)

        def _wait_g(j, rows, sem):
            pltpu.make_async_copy(t_hbm.at[srcv.at[pl.ds(j * K, K)]], rows,
                                  sem).wait()

        def _dload(j, dk, semd):
            pltpu.async_copy(dst_hbm.at[pl.ds(dbase + j * K, K)], dk, semd)

        def _wait_d(j, dk, semd):
            pltpu.make_async_copy(dst_hbm.at[pl.ds(dbase + j * K, K)], dk,
                                  semd).wait()

        _dload(0, dstk0, semd0)
        _dload(1, dstk1, semd1)
        _gather(0, rows0, sem0)
        _gather(1, rows1, sem1)

        def _pair(jj, _):
            j = 2 * jj
            _wait_g(j, rows0, sem0)
            _wait_d(j, dstk0, semd0)
            pltpu.sync_copy(rows0, acc.at[dstk0], add=True)
            _dload(j + 2, dstk0, semd0)
            _gather(j + 2, rows0, sem0)
            _wait_g(j + 1, rows1, sem1)
            _wait_d(j + 1, dstk1, semd1)
            pltpu.sync_copy(rows1, acc.at[dstk1], add=True)
            _dload(j + 3, dstk1, semd1)
            _gather(j + 3, rows1, sem1)
            return 0
        lax.fori_loop(0, (nchunk - 2) // 2, _pair, 0)

        if nchunk % 2 == 0:
            _wait_g(nchunk - 2, rows0, sem0)
            _wait_d(nchunk - 2, dstk0, semd0)
            pltpu.sync_copy(rows0, acc.at[dstk0], add=True)
            _wait_g(nchunk - 1, rows1, sem1)
            _wait_d(nchunk - 1, dstk1, semd1)
            pltpu.sync_copy(rows1, acc.at[dstk1], add=True)
        else:
            _wait_g(nchunk - 3, rows0, sem0)
            _wait_d(nchunk - 3, dstk0, semd0)
            pltpu.sync_copy(rows0, acc.at[dstk0], add=True)
            _dload(nchunk - 1, dstk0, semd0)
            _gather(nchunk - 1, rows0, sem0)
            _wait_g(nchunk - 2, rows1, sem1)
            _wait_d(nchunk - 2, dstk1, semd1)
            pltpu.sync_copy(rows1, acc.at[dstk1], add=True)
            _wait_g(nchunk - 1, rows0, sem0)
            _wait_d(nchunk - 1, dstk0, semd0)
            pltpu.sync_copy(rows0, acc.at[dstk0], add=True)

        plsc.subcore_barrier()
        pltpu.sync_copy(acc.at[pl.ds(s * RPT, RPT)],
                        z_hbm.at[pl.ds(c * NP + s * RPT, RPT)])

    return body


@functools.cache
def _get_sc_agg(dw, ept, split_edges):
    return pl.kernel(
        _make_agg_body(dw, ept, split_edges),
        out_type=jax.ShapeDtypeStruct((2 * NP, dw), jnp.float32),
        mesh=plsc.VectorSubcoreMesh(core_axis_name="c", subcore_axis_name="s",
                                    num_cores=NC, num_subcores=NS),
        scratch_types=[
            pltpu.VMEM((ept,), jnp.int32),
            pltpu.VMEM((K,), jnp.int32),
            pltpu.VMEM((K,), jnp.int32),
            pltpu.VMEM((K, dw), jnp.float32),
            pltpu.VMEM((K, dw), jnp.float32),
            pltpu.VMEM((ZROWS, dw), jnp.float32),
            pltpu.VMEM_SHARED((NP, dw), jnp.float32),
            pltpu.SemaphoreType.DMA,
            pltpu.SemaphoreType.DMA,
            pltpu.SemaphoreType.DMA,
            pltpu.SemaphoreType.DMA,
        ],
    )


def _sc_agg(t, src_flat, dst):
    return _get_sc_agg(D, EPT, False)(t, src_flat, dst)


def _sc_agg_deg(t, src_flat, dst):
    return _get_sc_agg(D, E // 2 // NS, True)(t, src_flat, dst)


def _sc_agg_final(t, src_flat, dst):
    return _get_sc_agg(D, E // 2 // NS, True)(t, src_flat, dst)


_B = 1000
_GRID = N // _B


def _scale_body(p0_ref, p1_ref, dinv_ref, dinv2_ref):
    deg = jnp.maximum(p0_ref[...] + p1_ref[...], 1.0)
    dinv2_ref[...] = 1.0 / deg
    dinv_ref[...] = lax.rsqrt(deg)


def _tc_scale(p0, p1):
    return pl.pallas_call(
        _scale_body,
        grid=(_GRID,),
        in_specs=[pl.BlockSpec((_B, D), lambda i: (i, 0)),
                  pl.BlockSpec((_B, D), lambda i: (i, 0))],
        out_specs=[pl.BlockSpec((_B, D), lambda i: (i, 0)),
                   pl.BlockSpec((_B, D), lambda i: (i, 0))],
        out_shape=[jax.ShapeDtypeStruct((N, D), jnp.float32),
                   jax.ShapeDtypeStruct((N, D), jnp.float32)],
    )(p0, p1)


def _layer0_body(x_ref, wr_ref, wi_ref, dinv_ref, gr_ref, gi_ref):
    x = x_ref[...]
    d = dinv_ref[...]
    gr_ref[...] = d * jnp.dot(x, wr_ref[...], preferred_element_type=jnp.float32)
    gi_ref[...] = d * jnp.dot(x, wi_ref[...], preferred_element_type=jnp.float32)


def _tc_layer0(x, wr, wi, dinv_b):
    return pl.pallas_call(
        _layer0_body,
        grid=(_GRID,),
        in_specs=[pl.BlockSpec((_B, D), lambda i: (i, 0)),
                  pl.BlockSpec((D, D), lambda i: (0, 0)),
                  pl.BlockSpec((D, D), lambda i: (0, 0)),
                  pl.BlockSpec((_B, D), lambda i: (i, 0))],
        out_specs=[pl.BlockSpec((_B, D), lambda i: (i, 0)),
                   pl.BlockSpec((_B, D), lambda i: (i, 0))],
        out_shape=[jax.ShapeDtypeStruct((N, D), jnp.float32),
                   jax.ShapeDtypeStruct((N, D), jnp.float32)],
    )(x, wr, wi, dinv_b)


def _layer_body(zr_ref, zi_ref, wr_ref, wi_ref, d2_ref, gr_ref, gi_ref):
    wr = wr_ref[...]
    wi = wi_ref[...]
    wra = 0.5 * (wr - wr.T)
    wia = 0.5 * (wi + wi.T)
    zr = zr_ref[...]
    zi = zi_ref[...]
    d2 = d2_ref[...]
    gr_ref[...] = d2 * (jnp.dot(zr, wra, preferred_element_type=jnp.float32)
                        - jnp.dot(zi, wia, preferred_element_type=jnp.float32))
    gi_ref[...] = d2 * (jnp.dot(zr, wia, preferred_element_type=jnp.float32)
                        + jnp.dot(zi, wra, preferred_element_type=jnp.float32))


def _tc_layer2(zr, zi, wr, wi, dinv2_b):
    return pl.pallas_call(
        _layer_body,
        grid=(_GRID,),
        in_specs=[pl.BlockSpec((_B, D), lambda i: (i, 0)),
                  pl.BlockSpec((_B, D), lambda i: (i, 0)),
                  pl.BlockSpec((D, D), lambda i: (0, 0)),
                  pl.BlockSpec((D, D), lambda i: (0, 0)),
                  pl.BlockSpec((_B, D), lambda i: (i, 0))],
        out_specs=[pl.BlockSpec((_B, D), lambda i: (i, 0)),
                   pl.BlockSpec((_B, D), lambda i: (i, 0))],
        out_shape=[jax.ShapeDtypeStruct((N, D), jnp.float32),
                   jax.ShapeDtypeStruct((N, D), jnp.float32)],
    )(zr, zi, wr, wi, dinv2_b)


def _final_body(z0_ref, z1_ref, dinv_ref, w_ref, b_ref, out_ref):
    xr = dinv_ref[...] * (z0_ref[...] + z1_ref[...])
    out_ref[...] = (jnp.dot(xr, w_ref[...], preferred_element_type=jnp.float32)
                    + b_ref[...])


def _tc_final(z0, z1, dinv_b, out_w, out_b2d):
    return pl.pallas_call(
        _final_body,
        grid=(_GRID,),
        in_specs=[pl.BlockSpec((_B, D), lambda i: (i, 0)),
                  pl.BlockSpec((_B, D), lambda i: (i, 0)),
                  pl.BlockSpec((_B, D), lambda i: (i, 0)),
                  pl.BlockSpec((D, D), lambda i: (0, 0)),
                  pl.BlockSpec((1, D), lambda i: (0, 0))],
        out_specs=pl.BlockSpec((_B, D), lambda i: (i, 0)),
        out_shape=jax.ShapeDtypeStruct((N, D), jnp.float32),
    )(z0, z1, dinv_b, out_w, out_b2d)


def kernel(x, edge_index, conv_Wr, conv_Wi, out_W, out_b):
    src = edge_index[0]
    dst = edge_index[1]
    src_flat = jnp.concatenate([src, src + NP])
    pad = jnp.zeros((NP - N, D), jnp.float32)
    ones_t = jnp.ones((N, D), jnp.float32)
    zdeg = _sc_agg_deg(ones_t, src_flat, dst)
    dinv_b, dinv2_b = _tc_scale(zdeg[:N], zdeg[NP:NP + N])

    gr, gi = _tc_layer0(x, conv_Wr[0], conv_Wi[0], dinv_b)
    nlayers = conv_Wr.shape[0]
    for i in range(1, nlayers):
        t = jnp.concatenate([gr, pad, gi], axis=0)
        z = _sc_agg(t, src_flat, dst)
        gr, gi = _tc_layer2(z[:N], z[NP:NP + N], conv_Wr[i], conv_Wi[i], dinv2_b)

    t = jnp.concatenate([gr, pad, gi], axis=0)
    z = _sc_agg_final(t, src_flat, dst)
    return _tc_final(z[:N], z[NP:NP + N], dinv_b, out_W, out_b.reshape(1, D))
